# baseline (speedup 1.0000x reference)
import functools

import jax
import jax.numpy as jnp
from jax import lax
from jax.experimental import pallas as pl
from jax.experimental.pallas import tpu as pltpu
from jax.experimental.pallas import tpu_sc as plsc

K_NN = 16
N_PTS = 8000
NPAD = 8192
ROWS = 512


def _knn_body(xa_ref, xbt_ref, xxr_ref, xxc_ref, out_ref):
    m = jnp.dot(xa_ref[...].astype(jnp.bfloat16),
                xbt_ref[...].astype(jnp.bfloat16),
                preferred_element_type=jnp.float32)
    inner = -2.0 * m
    s = (-xxr_ref[:, 0:1] - inner) - xxc_ref[0:1, :]
    col = lax.broadcasted_iota(jnp.int32, (ROWS, NPAD), 1)
    neg = jnp.float32(-jnp.inf)
    s = jnp.where(col < N_PTS, s, neg)
    for t in range(K_NN + 1):
        idx = jnp.argmax(s, axis=1).astype(jnp.int32)[:, None]
        out_ref[:, t : t + 1] = idx
        s = jnp.where(col == idx, neg, s)


def _knn_pallas(xa, xbt, xxr, xxc):
    return pl.pallas_call(
        _knn_body,
        grid=(NPAD // ROWS,),
        in_specs=[
            pl.BlockSpec((ROWS, 128), lambda i: (i, 0)),
            pl.BlockSpec((128, NPAD), lambda i: (0, 0)),
            pl.BlockSpec((ROWS, 8), lambda i: (i, 0)),
            pl.BlockSpec((8, NPAD), lambda i: (0, 0)),
        ],
        out_specs=pl.BlockSpec((ROWS, 32), lambda i: (i, 0)),
        out_shape=jax.ShapeDtypeStruct((NPAD, 32), jnp.int32),
    )(xa, xbt, xxr, xxc)


def _knn_idx(xt):
    n, c = xt.shape
    xx = jnp.sum(xt.T[None] * xt.T[None], axis=1)[0]
    xpad = jnp.pad(xt, ((0, NPAD - n), (0, 128 - c)))
    xxp = jnp.pad(xx, (0, NPAD - n))
    xxr = jnp.broadcast_to(xxp[:, None], (NPAD, 8))
    xxc = jnp.broadcast_to(xxp[None, :], (8, NPAD))
    out = _knn_pallas(xpad, xpad.T, xxr, xxc)
    return out[:N_PTS, 1 : K_NN + 1]


def _sc_gather3(t1, t2, t3, idx_flat):
    b = idx_flat.shape[0]
    info = plsc.get_sparse_core_info()
    nw = info.num_cores * info.num_subcores
    b_per_w = b // nw
    ch = 80
    n_iter = b_per_w // ch
    mesh = plsc.VectorSubcoreMesh(core_axis_name="c", subcore_axis_name="s")
    d1, d2, d3 = t1.shape[1], t2.shape[1], t3.shape[1]

    @functools.partial(
        pl.kernel,
        mesh=mesh,
        compiler_params=pltpu.CompilerParams(use_tc_tiling_on_sc=False),
        out_type=(
            jax.ShapeDtypeStruct((b, d1), jnp.float32),
            jax.ShapeDtypeStruct((b, d2), jnp.float32),
            jax.ShapeDtypeStruct((b, d3), jnp.float32),
        ),
        scratch_types=[
            pltpu.VMEM((ch,), jnp.int32),
            pltpu.VMEM((ch, d1), jnp.float32),
            pltpu.VMEM((ch, d2), jnp.float32),
            pltpu.VMEM((ch, d3), jnp.float32),
            pltpu.SemaphoreType.DMA,
        ],
    )
    def gat(t1_hbm, t2_hbm, t3_hbm, idx_hbm, o1_hbm, o2_hbm, o3_hbm,
            idx_v, r1_v, r2_v, r3_v, sem):
        wid = lax.axis_index("s") * info.num_cores + lax.axis_index("c")
        base = wid * b_per_w

        def body(j, carry):
            off = base + j * ch
            pltpu.sync_copy(idx_hbm.at[pl.ds(off, ch)], idx_v)
            c1 = pltpu.async_copy(t1_hbm.at[idx_v], r1_v, sem)
            c2 = pltpu.async_copy(t2_hbm.at[idx_v], r2_v, sem)
            c3 = pltpu.async_copy(t3_hbm.at[idx_v], r3_v, sem)
            c1.wait()
            c2.wait()
            c3.wait()
            pltpu.sync_copy(r1_v, o1_hbm.at[pl.ds(off, ch)])
            pltpu.sync_copy(r2_v, o2_hbm.at[pl.ds(off, ch)])
            pltpu.sync_copy(r3_v, o3_hbm.at[pl.ds(off, ch)])
            return carry

        lax.fori_loop(0, n_iter, body, 0)

    return gat(t1, t2, t3, idx_flat)


def _lrelu(x):
    return jax.nn.leaky_relu(x, 0.2)


def _bn(x):
    axes = (0,) + tuple(range(2, x.ndim))
    m = jnp.mean(x, axis=axes, keepdims=True)
    v = jnp.var(x, axis=axes, keepdims=True)
    return (x - m) * lax.rsqrt(v + 1e-5)


def _conv1d(x, w, bias=None):
    y = jnp.einsum('oc,bcn->bon', w, x)
    if bias is not None:
        y = y + bias[None, :, None]
    return y


def _stnkd(x, p, pre, k):
    h = jax.nn.relu(_bn(_conv1d(x, p[pre + '_c1'], p[pre + '_c1b'])))
    h = jax.nn.relu(_bn(_conv1d(h, p[pre + '_c2'], p[pre + '_c2b'])))
    h = jax.nn.relu(_bn(_conv1d(h, p[pre + '_c3'], p[pre + '_c3b'])))
    h = jnp.max(h, axis=2)
    h = jax.nn.relu(h @ p[pre + '_fc1'].T + p[pre + '_fc1b'])
    h = jax.nn.relu(h @ p[pre + '_fc2'].T + p[pre + '_fc2b'])
    h = h @ p[pre + '_fc3'].T + p[pre + '_fc3b']
    h = h + jnp.eye(k, dtype=h.dtype).reshape(1, k * k)
    return h.reshape(-1, k, k)


def _mmb(x, w):
    return jnp.dot(x.astype(jnp.bfloat16), w.astype(jnp.bfloat16),
                   preferred_element_type=jnp.float32)


def _edge_conv_stats(gc, gx, gn, tc, tx, tn, w1nb, w1cen, w2nb, w2cen, w3l, w3r):
    k = K_NN
    n = N_PTS
    p = 2000
    cc = gc.shape[2]
    cn = gn.shape[2]
    co1, co2, co3 = w1nb.shape[1], w2nb.shape[1], w3l.shape[1]

    def body(gc_r, gx_r, gn_r, tc_r, tx_r, tn_r, w1a_r, w1b_r, w2a_r, w2b_r,
             w3a_r, w3b_r, y1_r, y2_r, y3_r, s1_r, q1_r, s2_r, q2_r, s3_r, q3_r):
        gcb, gxb, gnb = gc_r[0], gx_r[0], gn_r[0]
        y1 = _mmb(gcb, w1a_r[...]) + _mmb(tc_r[...], w1b_r[...])
        y2 = _mmb(gnb, w2a_r[...]) + _mmb(tn_r[...], w2b_r[...])
        delta = tx_r[...] - gxb
        y3 = _mmb(delta, w3a_r[...]) + _mmb(gxb, w3b_r[...])
        y1_r[0] = y1
        y2_r[0] = y2
        y3_r[0] = y3
        first = (pl.program_id(0) == 0) & (pl.program_id(1) == 0)

        @pl.when(first)
        def _():
            s1_r[...] = jnp.zeros_like(s1_r)
            q1_r[...] = jnp.zeros_like(q1_r)
            s2_r[...] = jnp.zeros_like(s2_r)
            q2_r[...] = jnp.zeros_like(q2_r)
            s3_r[...] = jnp.zeros_like(s3_r)
            q3_r[...] = jnp.zeros_like(q3_r)

        for y, s_r, q_r in ((y1, s1_r, q1_r), (y2, s2_r, q2_r), (y3, s3_r, q3_r)):
            s_r[0:1, :] = s_r[0:1, :] + jnp.sum(y, axis=0, keepdims=True)
            q_r[0:1, :] = q_r[0:1, :] + jnp.sum(y * y, axis=0, keepdims=True)

    f32 = jnp.float32
    return pl.pallas_call(
        body,
        grid=(k, n // p),
        in_specs=[
            pl.BlockSpec((1, p, cc), lambda a, i: (a, i, 0)),
            pl.BlockSpec((1, p, cc), lambda a, i: (a, i, 0)),
            pl.BlockSpec((1, p, cn), lambda a, i: (a, i, 0)),
            pl.BlockSpec((p, cc), lambda a, i: (i, 0)),
            pl.BlockSpec((p, cc), lambda a, i: (i, 0)),
            pl.BlockSpec((p, cn), lambda a, i: (i, 0)),
            pl.BlockSpec((cc, co1), lambda a, i: (0, 0)),
            pl.BlockSpec((cc, co1), lambda a, i: (0, 0)),
            pl.BlockSpec((cn, co2), lambda a, i: (0, 0)),
            pl.BlockSpec((cn, co2), lambda a, i: (0, 0)),
            pl.BlockSpec((cc, co3), lambda a, i: (0, 0)),
            pl.BlockSpec((cc, co3), lambda a, i: (0, 0)),
        ],
        out_specs=[
            pl.BlockSpec((1, p, co1), lambda a, i: (a, i, 0)),
            pl.BlockSpec((1, p, co2), lambda a, i: (a, i, 0)),
            pl.BlockSpec((1, p, co3), lambda a, i: (a, i, 0)),
            pl.BlockSpec((8, co1), lambda a, i: (0, 0)),
            pl.BlockSpec((8, co1), lambda a, i: (0, 0)),
            pl.BlockSpec((8, co2), lambda a, i: (0, 0)),
            pl.BlockSpec((8, co2), lambda a, i: (0, 0)),
            pl.BlockSpec((8, co3), lambda a, i: (0, 0)),
            pl.BlockSpec((8, co3), lambda a, i: (0, 0)),
        ],
        out_shape=[
            jax.ShapeDtypeStruct((k, n, co1), f32),
            jax.ShapeDtypeStruct((k, n, co2), f32),
            jax.ShapeDtypeStruct((k, n, co3), f32),
            jax.ShapeDtypeStruct((8, co1), f32),
            jax.ShapeDtypeStruct((8, co1), f32),
            jax.ShapeDtypeStruct((8, co2), f32),
            jax.ShapeDtypeStruct((8, co2), f32),
            jax.ShapeDtypeStruct((8, co3), f32),
            jax.ShapeDtypeStruct((8, co3), f32),
        ],
    )(gc, gx, gn, tc, tx, tn, w1nb, w1cen, w2nb, w2cen, w3l, w3r)


def _edge_attn(y1, y2, y3, mu1, inv1, mu2, inv2, mu3, inv3):
    k = K_NN
    n = N_PTS
    p = 400
    co1 = y1.shape[2]
    co2 = y2.shape[2]

    def body(y1_r, y2_r, y3_r, m1_r, i1_r, m2_r, i2_r, m3_r, i3_r,
             co_r, no_r, wb, fb):
        m1, i1 = m1_r[0:1, :], i1_r[0:1, :]
        m2, i2 = m2_r[0:1, :], i2_r[0:1, :]
        m3, i3 = m3_r[0:1, :], i3_r[0:1, :]
        emax = None
        nmax = None
        for a in range(K_NN):
            e = jax.nn.leaky_relu((y3_r[a] - m3) * i3, 0.2)
            wb[a] = e
            emax = e if a == 0 else jnp.maximum(emax, e)
            fb[a] = jax.nn.leaky_relu((y1_r[a] - m1) * i1, 0.2)
            nk = jax.nn.leaky_relu((y2_r[a] - m2) * i2, 0.2)
            nmax = nk if a == 0 else jnp.maximum(nmax, nk)
        den = jnp.zeros_like(emax)
        for a in range(K_NN):
            wv = jnp.exp(wb[a] - emax)
            wb[a] = wv
            den = den + wv
        acc = jnp.zeros_like(emax)
        for a in range(K_NN):
            acc = acc + (wb[a] / den) * fb[a]
        co_r[...] = acc
        no_r[...] = nmax

    f32 = jnp.float32
    return pl.pallas_call(
        body,
        grid=(n // p,),
        in_specs=[
            pl.BlockSpec((k, p, co1), lambda i: (0, i, 0)),
            pl.BlockSpec((k, p, co2), lambda i: (0, i, 0)),
            pl.BlockSpec((k, p, co1), lambda i: (0, i, 0)),
            pl.BlockSpec((8, co1), lambda i: (0, 0)),
            pl.BlockSpec((8, co1), lambda i: (0, 0)),
            pl.BlockSpec((8, co2), lambda i: (0, 0)),
            pl.BlockSpec((8, co2), lambda i: (0, 0)),
            pl.BlockSpec((8, co1), lambda i: (0, 0)),
            pl.BlockSpec((8, co1), lambda i: (0, 0)),
        ],
        out_specs=[
            pl.BlockSpec((p, co1), lambda i: (i, 0)),
            pl.BlockSpec((p, co2), lambda i: (i, 0)),
        ],
        out_shape=[
            jax.ShapeDtypeStruct((n, co1), f32),
            jax.ShapeDtypeStruct((n, co2), f32),
        ],
        scratch_shapes=[
            pltpu.VMEM((k, p, co1), f32),
            pltpu.VMEM((k, p, co1), f32),
        ],
    )(y1, y2, y3, mu1, inv1, mu2, inv2, mu3, inv3)


def _mu_inv(s, q, m=float(N_PTS * K_NN)):
    mu = s[0:1, :] / m
    var = q[0:1, :] / m - mu * mu
    inv = lax.rsqrt(var + 1e-5)
    return (jnp.broadcast_to(mu, (8,) + mu.shape[1:]),
            jnp.broadcast_to(inv, (8,) + inv.shape[1:]))


def _act(v, act):
    if act == 'relu':
        return jax.nn.relu(v)
    if act == 'lrelu':
        return jax.nn.leaky_relu(v, 0.2)
    return v


def _mm_stats(xs, ws, nrms=None, act=None, bias=None, p=1000):
    n = xs[0].shape[0]
    co = ws[0].shape[1]
    nx = len(xs)
    f32 = jnp.float32

    def body(*refs):
        x_rs = refs[:nx]
        pos = nx
        n_rs = None
        if nrms is not None:
            n_rs = refs[pos:pos + 2 * nx]
            pos += 2 * nx
        w_rs = refs[pos:pos + nx]
        pos += nx
        b_r = refs[pos] if bias is not None else None
        pos += 1 if bias is not None else 0
        y_r, s_r, q_r = refs[pos:pos + 3]
        y = None
        for i in range(nx):
            xv = x_rs[i][...]
            if n_rs is not None:
                xv = _act((xv - n_rs[2 * i][0:1, :]) * n_rs[2 * i + 1][0:1, :], act)
            t = _mmb(xv, w_rs[i][...])
            y = t if y is None else y + t
        if b_r is not None:
            y = y + b_r[0:1, :]
        y_r[...] = y

        @pl.when(pl.program_id(0) == 0)
        def _():
            s_r[...] = jnp.zeros_like(s_r)
            q_r[...] = jnp.zeros_like(q_r)

        s_r[0:1, :] = s_r[0:1, :] + jnp.sum(y, axis=0, keepdims=True)
        q_r[0:1, :] = q_r[0:1, :] + jnp.sum(y * y, axis=0, keepdims=True)

    in_specs = [pl.BlockSpec((p, x.shape[1]), lambda i: (i, 0)) for x in xs]
    args = list(xs)
    if nrms is not None:
        for mu, inv in nrms:
            args += [mu, inv]
            in_specs += [pl.BlockSpec(mu.shape, lambda i: (0, 0)),
                         pl.BlockSpec(inv.shape, lambda i: (0, 0))]
    args += list(ws)
    in_specs += [pl.BlockSpec(w.shape, lambda i: (0, 0)) for w in ws]
    if bias is not None:
        args.append(bias)
        in_specs.append(pl.BlockSpec(bias.shape, lambda i: (0, 0)))
    return pl.pallas_call(
        body,
        grid=(n // p,),
        in_specs=in_specs,
        out_specs=[pl.BlockSpec((p, co), lambda i: (i, 0)),
                   pl.BlockSpec((8, co), lambda i: (0, 0)),
                   pl.BlockSpec((8, co), lambda i: (0, 0))],
        out_shape=[jax.ShapeDtypeStruct((n, co), f32),
                   jax.ShapeDtypeStruct((8, co), f32),
                   jax.ShapeDtypeStruct((8, co), f32)],
    )(*args)


def _colmax(x, mu, inv, p=1000):
    n, c = x.shape

    def body(x_r, mu_r, inv_r, o_r):
        v = jax.nn.relu((x_r[...] - mu_r[0:1, :]) * inv_r[0:1, :])
        m = jnp.max(v, axis=0, keepdims=True)

        @pl.when(pl.program_id(0) == 0)
        def _():
            o_r[...] = jnp.full_like(o_r, -jnp.inf)

        o_r[...] = jnp.maximum(o_r[...], m)

    return pl.pallas_call(
        body,
        grid=(n // p,),
        in_specs=[pl.BlockSpec((p, c), lambda i: (i, 0)),
                  pl.BlockSpec((8, c), lambda i: (0, 0)),
                  pl.BlockSpec((8, c), lambda i: (0, 0))],
        out_specs=pl.BlockSpec((8, c), lambda i: (0, 0)),
        out_shape=jax.ShapeDtypeStruct((8, c), jnp.float32),
    )(x, mu, inv)


def _stn_fc(h, fc1, b1, fc2, b2, fc3, b3, eyef):
    def body(h_r, w1_r, b1_r, w2_r, b2_r, w3_r, b3_r, e_r, o_r):
        v = jax.nn.relu(_mmb(h_r[...], w1_r[...]) + b1_r[0:1, :])
        v = jax.nn.relu(_mmb(v, w2_r[...]) + b2_r[0:1, :])
        v = (_mmb(v, w3_r[...]) + b3_r[0:1, :]) + e_r[0:1, :]
        o_r[...] = v

    return pl.pallas_call(
        body,
        in_specs=[pl.BlockSpec(a.shape, lambda: (0,) * a.ndim)
                  for a in (h, fc1, b1, fc2, b2, fc3, b3, eyef)],
        out_specs=pl.BlockSpec((8, 144), lambda: (0, 0)),
        out_shape=jax.ShapeDtypeStruct((8, 144), jnp.float32),
    )(h, fc1, b1, fc2, b2, fc3, b3, eyef)


def _apply_trans(xt, trans, p=1000):
    n, c = xt.shape

    def body(x_r, t_r, o_r):
        o_r[...] = _mmb(x_r[...], t_r[...])

    return pl.pallas_call(
        body,
        grid=(n // p,),
        in_specs=[pl.BlockSpec((p, c), lambda i: (i, 0)),
                  pl.BlockSpec(trans.shape, lambda i: (0, 0))],
        out_specs=pl.BlockSpec((p, c), lambda i: (i, 0)),
        out_shape=jax.ShapeDtypeStruct((n, c), jnp.float32),
    )(xt, trans)


def _stn_point_major(xt, p, pre):
    w1 = jnp.pad(p[pre + '_c1'].T, ((0, 4), (0, 0)))
    y1, s, q = _mm_stats([xt], [w1], bias=_brow(p[pre + '_c1b']))
    y2, s2, q2 = _mm_stats([y1], [p[pre + '_c2'].T], nrms=[_mu_inv(s, q, 8000.0)],
                           act='relu', bias=_brow(p[pre + '_c2b']))
    y3, s3, q3 = _mm_stats([y2], [p[pre + '_c3'].T], nrms=[_mu_inv(s2, q2, 8000.0)],
                           act='relu', bias=_brow(p[pre + '_c3b']))
    h = _colmax(y3, *_mu_inv(s3, q3, 8000.0))
    eyef = _brow(jnp.eye(12, dtype=jnp.float32).reshape(144))
    t = _stn_fc(h, p[pre + '_fc1'].T, _brow(p[pre + '_fc1b']),
                p[pre + '_fc2'].T, _brow(p[pre + '_fc2b']),
                p[pre + '_fc3'].T, _brow(p[pre + '_fc3b']), eyef)
    trans = jnp.pad(t[0].reshape(12, 12), ((0, 4), (0, 4)))
    return _apply_trans(xt, trans)


def _brow(v):
    return jnp.broadcast_to(v[None, :], (8, v.shape[0]))


def _head_mid(y4c, y4n, mu_c, inv_c, mu_n, inv_n, fa_l, fa_r, p=1000):
    n = y4c.shape[0]
    f32 = jnp.float32

    def body(yc_r, yn_r, mc_r, ic_r, mn_r, in_r, fl_r, fr_r,
             hc_r, hn_r, yfa_r, s_r, q_r):
        ca = jax.nn.leaky_relu((yc_r[...] - mc_r[0:1, :]) * ic_r[0:1, :], 0.2)
        na = jax.nn.leaky_relu((yn_r[...] - mn_r[0:1, :]) * in_r[0:1, :], 0.2)
        avg_c = jnp.sum(ca, axis=1, keepdims=True) / 512.0
        avg_n = jnp.sum(na, axis=1, keepdims=True) / 512.0
        avg = avg_c + avg_n
        hc = ca * (avg_c / avg)
        hn = na * (avg_n / avg)
        hc_r[...] = hc
        hn_r[...] = hn
        yfa = _mmb(hc, fl_r[...]) + _mmb(hn, fr_r[...])
        yfa_r[...] = yfa

        @pl.when(pl.program_id(0) == 0)
        def _():
            s_r[...] = jnp.zeros_like(s_r)
            q_r[...] = jnp.zeros_like(q_r)

        s_r[0:1, :] = s_r[0:1, :] + jnp.sum(yfa, axis=0, keepdims=True)
        q_r[0:1, :] = q_r[0:1, :] + jnp.sum(yfa * yfa, axis=0, keepdims=True)

    return pl.pallas_call(
        body,
        grid=(n // p,),
        in_specs=[pl.BlockSpec((p, 256), lambda i: (i, 0)),
                  pl.BlockSpec((p, 256), lambda i: (i, 0))] +
                 [pl.BlockSpec((8, 256), lambda i: (0, 0))] * 4 +
                 [pl.BlockSpec((256, 512), lambda i: (0, 0))] * 2,
        out_specs=[pl.BlockSpec((p, 256), lambda i: (i, 0)),
                   pl.BlockSpec((p, 256), lambda i: (i, 0)),
                   pl.BlockSpec((p, 512), lambda i: (i, 0)),
                   pl.BlockSpec((8, 512), lambda i: (0, 0)),
                   pl.BlockSpec((8, 512), lambda i: (0, 0))],
        out_shape=[jax.ShapeDtypeStruct((n, 256), f32),
                   jax.ShapeDtypeStruct((n, 256), f32),
                   jax.ShapeDtypeStruct((n, 512), f32),
                   jax.ShapeDtypeStruct((8, 512), f32),
                   jax.ShapeDtypeStruct((8, 512), f32)],
    )(y4c, y4n, mu_c, inv_c, mu_n, inv_n, fa_l, fa_r)


def _head_fa(hc, hn, yfa, mu, inv, p1l, p1r, p=1000):
    n = hc.shape[0]
    f32 = jnp.float32

    def body(hc_r, hn_r, yfa_r, m_r, i_r, wl_r, wr_r, y_r, s_r, q_r):
        w = jax.nn.leaky_relu((yfa_r[...] - m_r[0:1, :]) * i_r[0:1, :], 0.2)
        h2c = w[:, 0:256] * hc_r[...]
        h2n = w[:, 256:512] * hn_r[...]
        y = _mmb(h2c, wl_r[...]) + _mmb(h2n, wr_r[...])
        y_r[...] = y

        @pl.when(pl.program_id(0) == 0)
        def _():
            s_r[...] = jnp.zeros_like(s_r)
            q_r[...] = jnp.zeros_like(q_r)

        s_r[0:1, :] = s_r[0:1, :] + jnp.sum(y, axis=0, keepdims=True)
        q_r[0:1, :] = q_r[0:1, :] + jnp.sum(y * y, axis=0, keepdims=True)

    return pl.pallas_call(
        body,
        grid=(n // p,),
        in_specs=[pl.BlockSpec((p, 256), lambda i: (i, 0)),
                  pl.BlockSpec((p, 256), lambda i: (i, 0)),
                  pl.BlockSpec((p, 512), lambda i: (i, 0)),
                  pl.BlockSpec((8, 512), lambda i: (0, 0)),
                  pl.BlockSpec((8, 512), lambda i: (0, 0)),
                  pl.BlockSpec((256, 512), lambda i: (0, 0)),
                  pl.BlockSpec((256, 512), lambda i: (0, 0))],
        out_specs=[pl.BlockSpec((p, 512), lambda i: (i, 0)),
                   pl.BlockSpec((8, 512), lambda i: (0, 0)),
                   pl.BlockSpec((8, 512), lambda i: (0, 0))],
        out_shape=[jax.ShapeDtypeStruct((n, 512), f32),
                   jax.ShapeDtypeStruct((8, 512), f32),
                   jax.ShapeDtypeStruct((8, 512), f32)],
    )(hc, hn, yfa, mu, inv, p1l, p1r)


def _head_final(y7, mu, inv, w4p, p=1000):
    n = y7.shape[0]

    def body(y_r, m_r, i_r, w_r, o_r):
        a = jax.nn.leaky_relu((y_r[...] - m_r[0:1, :]) * i_r[0:1, :], 0.2)
        yv = _mmb(a, w_r[...])
        s0 = yv[:, 0:1]
        s1 = yv[:, 1:2]
        mx = jnp.maximum(s0, s1)
        e0 = s0 - mx
        e1 = s1 - mx
        lse = jnp.log(jnp.exp(e0) + jnp.exp(e1))
        o_r[...] = jnp.concatenate([e0 - lse, e1 - lse], axis=1)

    return pl.pallas_call(
        body,
        grid=(n // p,),
        in_specs=[pl.BlockSpec((p, 128), lambda i: (i, 0)),
                  pl.BlockSpec((8, 128), lambda i: (0, 0)),
                  pl.BlockSpec((8, 128), lambda i: (0, 0)),
                  pl.BlockSpec((128, 128), lambda i: (0, 0))],
        out_specs=pl.BlockSpec((p, 2), lambda i: (i, 0)),
        out_shape=jax.ShapeDtypeStruct((n, 2), jnp.float32),
    )(y7, mu, inv, w4p)


def _pad_lanes(a, width):
    return jnp.pad(a, ((0, 0), (0, width - a.shape[1])))


def _edge_layer(ct, xr, tn, wconv_c, wconv_n, watt, c, cpad):
    idx = _knn_idx(ct)
    idx_t = idx.T.reshape(N_PTS * K_NN)
    gc, gx, gn = _sc_gather3(ct, xr, tn, idx_t)
    k3 = (K_NN, N_PTS, cpad)
    gc, gx, gn = gc.reshape(k3), gx.reshape(k3), gn.reshape(k3)

    def split(w):
        wl = jnp.pad(w[:, :c].T, ((0, cpad - c), (0, 0)))
        wr = jnp.pad(w[:, c:].T, ((0, cpad - c), (0, 0)))
        return wl, wr
    w1nb, w1cen = split(wconv_c)
    w2nb, w2cen = split(wconv_n)
    w3l, w3r = split(watt)

    y1, y2, y3, s1, q1, s2, q2, s3, q3 = _edge_conv_stats(
        gc, gx, gn, ct, xr, tn, w1nb, w1cen, w2nb, w2cen, w3l, w3r)
    mu1, inv1 = _mu_inv(s1, q1)
    mu2, inv2 = _mu_inv(s2, q2)
    mu3, inv3 = _mu_inv(s3, q3)
    return _edge_attn(y1, y2, y3, mu1, inv1, mu2, inv2, mu3, inv3)


def kernel(x, att1_c, att2_c, att3_c, conv1_c, conv1_n, conv2_c, conv2_n,
           conv3_c, conv3_n, conv4_c, conv4_n, fa, pred1, pred2, pred3, pred4,
           stnc_c1, stnc_c1b, stnc_c2, stnc_c2b, stnc_c3, stnc_c3b,
           stnc_fc1, stnc_fc1b, stnc_fc2, stnc_fc2b, stnc_fc3, stnc_fc3b,
           stnn_c1, stnn_c1b, stnn_c2, stnn_c2b, stnn_c3, stnn_c3b,
           stnn_fc1, stnn_fc1b, stnn_fc2, stnn_fc2b, stnn_fc3, stnn_fc3b):
    p = dict(
        stnc_c1=stnc_c1, stnc_c1b=stnc_c1b, stnc_c2=stnc_c2, stnc_c2b=stnc_c2b,
        stnc_c3=stnc_c3, stnc_c3b=stnc_c3b, stnc_fc1=stnc_fc1,
        stnc_fc1b=stnc_fc1b, stnc_fc2=stnc_fc2, stnc_fc2b=stnc_fc2b,
        stnc_fc3=stnc_fc3, stnc_fc3b=stnc_fc3b,
        stnn_c1=stnn_c1, stnn_c1b=stnn_c1b, stnn_c2=stnn_c2, stnn_c2b=stnn_c2b,
        stnn_c3=stnn_c3, stnn_c3b=stnn_c3b, stnn_fc1=stnn_fc1,
        stnn_fc1b=stnn_fc1b, stnn_fc2=stnn_fc2, stnn_fc2b=stnn_fc2b,
        stnn_fc3=stnn_fc3, stnn_fc3b=stnn_fc3b,
    )
    B, _, N = x.shape
    coor = x[:, :12, :]
    nor = x[:, 12:, :]
    trans_c = _stnkd(coor, p, 'stnc', 12)
    coor = jnp.einsum('bcn,bcd->bdn', coor, trans_c)
    trans_n = _stnkd(nor, p, 'stnn', 12)
    nor = jnp.einsum('bcn,bcd->bdn', nor, trans_n)
    ct1 = _pad_lanes(coor[0].T, 16)
    tn1 = _pad_lanes(nor[0].T, 16)
    xr1 = _pad_lanes(coor.reshape(N, 12), 16)
    co1, no1 = _edge_layer(ct1, xr1, tn1, conv1_c, conv1_n, att1_c, 12, 16)

    xr2 = co1.T.reshape(N, 32)
    co2, no2 = _edge_layer(co1, xr2, no1, conv2_c, conv2_n, att2_c, 32, 32)

    xr3 = co2.T.reshape(N, 64)
    co3, no3 = _edge_layer(co2, xr3, no2, conv3_c, conv3_n, att3_c, 64, 64)

    y4c, s4c, q4c = _mm_stats(
        [co1, co2, co3],
        [conv4_c[:, :32].T, conv4_c[:, 32:96].T, conv4_c[:, 96:].T])
    y4n, s4n, q4n = _mm_stats(
        [no1, no2, no3],
        [conv4_n[:, :32].T, conv4_n[:, 32:96].T, conv4_n[:, 96:].T])
    mu_c, inv_c = _mu_inv(s4c, q4c, 8000.0)
    mu_n, inv_n = _mu_inv(s4n, q4n, 8000.0)
    hc, hn, yfa, sfa, qfa = _head_mid(y4c, y4n, mu_c, inv_c, mu_n, inv_n,
                                      fa[:, :256].T, fa[:, 256:].T)
    mu_f, inv_f = _mu_inv(sfa, qfa, 8000.0)
    y5, s5, q5 = _head_fa(hc, hn, yfa, mu_f, inv_f,
                          pred1[:, :256].T, pred1[:, 256:].T)
    y6, s6, q6 = _mm_stats([y5], [pred2.T], nrms=[_mu_inv(s5, q5, 8000.0)],
                           act='lrelu')
    y7, s7, q7 = _mm_stats([y6], [pred3.T], nrms=[_mu_inv(s6, q6, 8000.0)],
                           act='lrelu')
    w4p = jnp.pad(pred4.T, ((0, 0), (0, 126)))
    out = _head_final(y7, *_mu_inv(s7, q7, 8000.0), w4p)
    return out[None]


# speedup vs baseline: 1.0613x; 1.0613x over previous
import functools

import jax
import jax.numpy as jnp
from jax import lax
from jax.experimental import pallas as pl
from jax.experimental.pallas import tpu as pltpu
from jax.experimental.pallas import tpu_sc as plsc

K_NN = 16
N_PTS = 8000
NPAD = 8192
ROWS = 128


def _knn_body(xa_ref, xbt_ref, xxr_ref, xxc_ref, out_ref):
    m = jnp.dot(xa_ref[...].astype(jnp.bfloat16),
                xbt_ref[...].astype(jnp.bfloat16),
                preferred_element_type=jnp.float32)
    inner = -2.0 * m
    s = (-xxr_ref[:, 0:1] - inner) - xxc_ref[0:1, :]
    col = lax.broadcasted_iota(jnp.int32, (ROWS, NPAD), 1)
    neg = jnp.float32(-jnp.inf)
    s = jnp.where(col < N_PTS, s, neg)
    for t in range(K_NN + 1):
        idx = jnp.argmax(s, axis=1).astype(jnp.int32)[:, None]
        out_ref[:, t : t + 1] = idx
        s = jnp.where(col == idx, neg, s)


def _knn_pallas(xa, xbt, xxr, xxc):
    return pl.pallas_call(
        _knn_body,
        grid=(NPAD // ROWS,),
        in_specs=[
            pl.BlockSpec((ROWS, 128), lambda i: (i, 0)),
            pl.BlockSpec((128, NPAD), lambda i: (0, 0)),
            pl.BlockSpec((ROWS, 8), lambda i: (i, 0)),
            pl.BlockSpec((8, NPAD), lambda i: (0, 0)),
        ],
        out_specs=pl.BlockSpec((ROWS, 32), lambda i: (i, 0)),
        out_shape=jax.ShapeDtypeStruct((NPAD, 32), jnp.int32),
    )(xa, xbt, xxr, xxc)


def _knn_idx(xt):
    n, c = xt.shape
    xx = jnp.sum(xt.T[None] * xt.T[None], axis=1)[0]
    xpad = jnp.pad(xt, ((0, NPAD - n), (0, 128 - c)))
    xxp = jnp.pad(xx, (0, NPAD - n))
    xxr = jnp.broadcast_to(xxp[:, None], (NPAD, 8))
    xxc = jnp.broadcast_to(xxp[None, :], (8, NPAD))
    out = _knn_pallas(xpad, xpad.T, xxr, xxc)
    return out[:N_PTS, 1 : K_NN + 1]


def _sc_gather3(t1, t2, t3, idx_flat):
    b = idx_flat.shape[0]
    info = plsc.get_sparse_core_info()
    nw = info.num_cores * info.num_subcores
    b_per_w = b // nw
    ch = 80
    n_iter = b_per_w // ch
    mesh = plsc.VectorSubcoreMesh(core_axis_name="c", subcore_axis_name="s")
    d1, d2, d3 = t1.shape[1], t2.shape[1], t3.shape[1]

    @functools.partial(
        pl.kernel,
        mesh=mesh,
        compiler_params=pltpu.CompilerParams(use_tc_tiling_on_sc=False),
        out_type=(
            jax.ShapeDtypeStruct((b, d1), jnp.float32),
            jax.ShapeDtypeStruct((b, d2), jnp.float32),
            jax.ShapeDtypeStruct((b, d3), jnp.float32),
        ),
        scratch_types=[
            pltpu.VMEM((ch,), jnp.int32),
            pltpu.VMEM((ch, d1), jnp.float32),
            pltpu.VMEM((ch, d2), jnp.float32),
            pltpu.VMEM((ch, d3), jnp.float32),
            pltpu.SemaphoreType.DMA,
        ],
    )
    def gat(t1_hbm, t2_hbm, t3_hbm, idx_hbm, o1_hbm, o2_hbm, o3_hbm,
            idx_v, r1_v, r2_v, r3_v, sem):
        wid = lax.axis_index("s") * info.num_cores + lax.axis_index("c")
        base = wid * b_per_w

        def body(j, carry):
            off = base + j * ch
            pltpu.sync_copy(idx_hbm.at[pl.ds(off, ch)], idx_v)
            c1 = pltpu.async_copy(t1_hbm.at[idx_v], r1_v, sem)
            c2 = pltpu.async_copy(t2_hbm.at[idx_v], r2_v, sem)
            c3 = pltpu.async_copy(t3_hbm.at[idx_v], r3_v, sem)
            c1.wait()
            c2.wait()
            c3.wait()
            pltpu.sync_copy(r1_v, o1_hbm.at[pl.ds(off, ch)])
            pltpu.sync_copy(r2_v, o2_hbm.at[pl.ds(off, ch)])
            pltpu.sync_copy(r3_v, o3_hbm.at[pl.ds(off, ch)])
            return carry

        lax.fori_loop(0, n_iter, body, 0)

    return gat(t1, t2, t3, idx_flat)


def _lrelu(x):
    return jax.nn.leaky_relu(x, 0.2)


def _bn(x):
    axes = (0,) + tuple(range(2, x.ndim))
    m = jnp.mean(x, axis=axes, keepdims=True)
    v = jnp.var(x, axis=axes, keepdims=True)
    return (x - m) * lax.rsqrt(v + 1e-5)


def _conv1d(x, w, bias=None):
    y = jnp.einsum('oc,bcn->bon', w, x)
    if bias is not None:
        y = y + bias[None, :, None]
    return y


def _stnkd(x, p, pre, k):
    h = jax.nn.relu(_bn(_conv1d(x, p[pre + '_c1'], p[pre + '_c1b'])))
    h = jax.nn.relu(_bn(_conv1d(h, p[pre + '_c2'], p[pre + '_c2b'])))
    h = jax.nn.relu(_bn(_conv1d(h, p[pre + '_c3'], p[pre + '_c3b'])))
    h = jnp.max(h, axis=2)
    h = jax.nn.relu(h @ p[pre + '_fc1'].T + p[pre + '_fc1b'])
    h = jax.nn.relu(h @ p[pre + '_fc2'].T + p[pre + '_fc2b'])
    h = h @ p[pre + '_fc3'].T + p[pre + '_fc3b']
    h = h + jnp.eye(k, dtype=h.dtype).reshape(1, k * k)
    return h.reshape(-1, k, k)


def _mmb(x, w):
    return jnp.dot(x.astype(jnp.bfloat16), w.astype(jnp.bfloat16),
                   preferred_element_type=jnp.float32)


def _edge_conv_stats(gc, gx, gn, tc, tx, tn, w1nb, w1cen, w2nb, w2cen, w3l, w3r):
    k = K_NN
    n = N_PTS
    p = 2000
    cc = gc.shape[2]
    cn = gn.shape[2]
    co1, co2, co3 = w1nb.shape[1], w2nb.shape[1], w3l.shape[1]

    def body(gc_r, gx_r, gn_r, tc_r, tx_r, tn_r, w1a_r, w1b_r, w2a_r, w2b_r,
             w3a_r, w3b_r, y1_r, y2_r, y3_r, s1_r, q1_r, s2_r, q2_r, s3_r, q3_r):
        gcb, gxb, gnb = gc_r[0], gx_r[0], gn_r[0]
        y1 = _mmb(gcb, w1a_r[...]) + _mmb(tc_r[...], w1b_r[...])
        y2 = _mmb(gnb, w2a_r[...]) + _mmb(tn_r[...], w2b_r[...])
        delta = tx_r[...] - gxb
        y3 = _mmb(delta, w3a_r[...]) + _mmb(gxb, w3b_r[...])
        y1_r[0] = y1
        y2_r[0] = y2
        y3_r[0] = y3
        first = (pl.program_id(0) == 0) & (pl.program_id(1) == 0)

        @pl.when(first)
        def _():
            s1_r[...] = jnp.zeros_like(s1_r)
            q1_r[...] = jnp.zeros_like(q1_r)
            s2_r[...] = jnp.zeros_like(s2_r)
            q2_r[...] = jnp.zeros_like(q2_r)
            s3_r[...] = jnp.zeros_like(s3_r)
            q3_r[...] = jnp.zeros_like(q3_r)

        for y, s_r, q_r in ((y1, s1_r, q1_r), (y2, s2_r, q2_r), (y3, s3_r, q3_r)):
            s_r[0:1, :] = s_r[0:1, :] + jnp.sum(y, axis=0, keepdims=True)
            q_r[0:1, :] = q_r[0:1, :] + jnp.sum(y * y, axis=0, keepdims=True)

    f32 = jnp.float32
    return pl.pallas_call(
        body,
        grid=(k, n // p),
        in_specs=[
            pl.BlockSpec((1, p, cc), lambda a, i: (a, i, 0)),
            pl.BlockSpec((1, p, cc), lambda a, i: (a, i, 0)),
            pl.BlockSpec((1, p, cn), lambda a, i: (a, i, 0)),
            pl.BlockSpec((p, cc), lambda a, i: (i, 0)),
            pl.BlockSpec((p, cc), lambda a, i: (i, 0)),
            pl.BlockSpec((p, cn), lambda a, i: (i, 0)),
            pl.BlockSpec((cc, co1), lambda a, i: (0, 0)),
            pl.BlockSpec((cc, co1), lambda a, i: (0, 0)),
            pl.BlockSpec((cn, co2), lambda a, i: (0, 0)),
            pl.BlockSpec((cn, co2), lambda a, i: (0, 0)),
            pl.BlockSpec((cc, co3), lambda a, i: (0, 0)),
            pl.BlockSpec((cc, co3), lambda a, i: (0, 0)),
        ],
        out_specs=[
            pl.BlockSpec((1, p, co1), lambda a, i: (a, i, 0)),
            pl.BlockSpec((1, p, co2), lambda a, i: (a, i, 0)),
            pl.BlockSpec((1, p, co3), lambda a, i: (a, i, 0)),
            pl.BlockSpec((8, co1), lambda a, i: (0, 0)),
            pl.BlockSpec((8, co1), lambda a, i: (0, 0)),
            pl.BlockSpec((8, co2), lambda a, i: (0, 0)),
            pl.BlockSpec((8, co2), lambda a, i: (0, 0)),
            pl.BlockSpec((8, co3), lambda a, i: (0, 0)),
            pl.BlockSpec((8, co3), lambda a, i: (0, 0)),
        ],
        out_shape=[
            jax.ShapeDtypeStruct((k, n, co1), f32),
            jax.ShapeDtypeStruct((k, n, co2), f32),
            jax.ShapeDtypeStruct((k, n, co3), f32),
            jax.ShapeDtypeStruct((8, co1), f32),
            jax.ShapeDtypeStruct((8, co1), f32),
            jax.ShapeDtypeStruct((8, co2), f32),
            jax.ShapeDtypeStruct((8, co2), f32),
            jax.ShapeDtypeStruct((8, co3), f32),
            jax.ShapeDtypeStruct((8, co3), f32),
        ],
    )(gc, gx, gn, tc, tx, tn, w1nb, w1cen, w2nb, w2cen, w3l, w3r)


def _edge_attn(y1, y2, y3, mu1, inv1, mu2, inv2, mu3, inv3):
    k = K_NN
    n = N_PTS
    p = 400
    co1 = y1.shape[2]
    co2 = y2.shape[2]

    def body(y1_r, y2_r, y3_r, m1_r, i1_r, m2_r, i2_r, m3_r, i3_r,
             co_r, no_r, wb, fb):
        m1, i1 = m1_r[0:1, :], i1_r[0:1, :]
        m2, i2 = m2_r[0:1, :], i2_r[0:1, :]
        m3, i3 = m3_r[0:1, :], i3_r[0:1, :]
        emax = None
        nmax = None
        for a in range(K_NN):
            e = jax.nn.leaky_relu((y3_r[a] - m3) * i3, 0.2)
            wb[a] = e
            emax = e if a == 0 else jnp.maximum(emax, e)
            fb[a] = jax.nn.leaky_relu((y1_r[a] - m1) * i1, 0.2)
            nk = jax.nn.leaky_relu((y2_r[a] - m2) * i2, 0.2)
            nmax = nk if a == 0 else jnp.maximum(nmax, nk)
        den = jnp.zeros_like(emax)
        for a in range(K_NN):
            wv = jnp.exp(wb[a] - emax)
            wb[a] = wv
            den = den + wv
        acc = jnp.zeros_like(emax)
        for a in range(K_NN):
            acc = acc + (wb[a] / den) * fb[a]
        co_r[...] = acc
        no_r[...] = nmax

    f32 = jnp.float32
    return pl.pallas_call(
        body,
        grid=(n // p,),
        in_specs=[
            pl.BlockSpec((k, p, co1), lambda i: (0, i, 0)),
            pl.BlockSpec((k, p, co2), lambda i: (0, i, 0)),
            pl.BlockSpec((k, p, co1), lambda i: (0, i, 0)),
            pl.BlockSpec((8, co1), lambda i: (0, 0)),
            pl.BlockSpec((8, co1), lambda i: (0, 0)),
            pl.BlockSpec((8, co2), lambda i: (0, 0)),
            pl.BlockSpec((8, co2), lambda i: (0, 0)),
            pl.BlockSpec((8, co1), lambda i: (0, 0)),
            pl.BlockSpec((8, co1), lambda i: (0, 0)),
        ],
        out_specs=[
            pl.BlockSpec((p, co1), lambda i: (i, 0)),
            pl.BlockSpec((p, co2), lambda i: (i, 0)),
        ],
        out_shape=[
            jax.ShapeDtypeStruct((n, co1), f32),
            jax.ShapeDtypeStruct((n, co2), f32),
        ],
        scratch_shapes=[
            pltpu.VMEM((k, p, co1), f32),
            pltpu.VMEM((k, p, co1), f32),
        ],
    )(y1, y2, y3, mu1, inv1, mu2, inv2, mu3, inv3)


def _mu_inv(s, q, m=float(N_PTS * K_NN)):
    mu = s[0:1, :] / m
    var = q[0:1, :] / m - mu * mu
    inv = lax.rsqrt(var + 1e-5)
    return (jnp.broadcast_to(mu, (8,) + mu.shape[1:]),
            jnp.broadcast_to(inv, (8,) + inv.shape[1:]))


def _act(v, act):
    if act == 'relu':
        return jax.nn.relu(v)
    if act == 'lrelu':
        return jax.nn.leaky_relu(v, 0.2)
    return v


def _mm_stats(xs, ws, nrms=None, act=None, bias=None, p=1000):
    n = xs[0].shape[0]
    co = ws[0].shape[1]
    nx = len(xs)
    f32 = jnp.float32

    def body(*refs):
        x_rs = refs[:nx]
        pos = nx
        n_rs = None
        if nrms is not None:
            n_rs = refs[pos:pos + 2 * nx]
            pos += 2 * nx
        w_rs = refs[pos:pos + nx]
        pos += nx
        b_r = refs[pos] if bias is not None else None
        pos += 1 if bias is not None else 0
        y_r, s_r, q_r = refs[pos:pos + 3]
        y = None
        for i in range(nx):
            xv = x_rs[i][...]
            if n_rs is not None:
                xv = _act((xv - n_rs[2 * i][0:1, :]) * n_rs[2 * i + 1][0:1, :], act)
            t = _mmb(xv, w_rs[i][...])
            y = t if y is None else y + t
        if b_r is not None:
            y = y + b_r[0:1, :]
        y_r[...] = y

        @pl.when(pl.program_id(0) == 0)
        def _():
            s_r[...] = jnp.zeros_like(s_r)
            q_r[...] = jnp.zeros_like(q_r)

        s_r[0:1, :] = s_r[0:1, :] + jnp.sum(y, axis=0, keepdims=True)
        q_r[0:1, :] = q_r[0:1, :] + jnp.sum(y * y, axis=0, keepdims=True)

    in_specs = [pl.BlockSpec((p, x.shape[1]), lambda i: (i, 0)) for x in xs]
    args = list(xs)
    if nrms is not None:
        for mu, inv in nrms:
            args += [mu, inv]
            in_specs += [pl.BlockSpec(mu.shape, lambda i: (0, 0)),
                         pl.BlockSpec(inv.shape, lambda i: (0, 0))]
    args += list(ws)
    in_specs += [pl.BlockSpec(w.shape, lambda i: (0, 0)) for w in ws]
    if bias is not None:
        args.append(bias)
        in_specs.append(pl.BlockSpec(bias.shape, lambda i: (0, 0)))
    return pl.pallas_call(
        body,
        grid=(n // p,),
        in_specs=in_specs,
        out_specs=[pl.BlockSpec((p, co), lambda i: (i, 0)),
                   pl.BlockSpec((8, co), lambda i: (0, 0)),
                   pl.BlockSpec((8, co), lambda i: (0, 0))],
        out_shape=[jax.ShapeDtypeStruct((n, co), f32),
                   jax.ShapeDtypeStruct((8, co), f32),
                   jax.ShapeDtypeStruct((8, co), f32)],
    )(*args)


def _colmax(x, mu, inv, p=1000):
    n, c = x.shape

    def body(x_r, mu_r, inv_r, o_r):
        v = jax.nn.relu((x_r[...] - mu_r[0:1, :]) * inv_r[0:1, :])
        m = jnp.max(v, axis=0, keepdims=True)

        @pl.when(pl.program_id(0) == 0)
        def _():
            o_r[...] = jnp.full_like(o_r, -jnp.inf)

        o_r[...] = jnp.maximum(o_r[...], m)

    return pl.pallas_call(
        body,
        grid=(n // p,),
        in_specs=[pl.BlockSpec((p, c), lambda i: (i, 0)),
                  pl.BlockSpec((8, c), lambda i: (0, 0)),
                  pl.BlockSpec((8, c), lambda i: (0, 0))],
        out_specs=pl.BlockSpec((8, c), lambda i: (0, 0)),
        out_shape=jax.ShapeDtypeStruct((8, c), jnp.float32),
    )(x, mu, inv)


def _stn_fc(h, fc1, b1, fc2, b2, fc3, b3, eyef):
    def body(h_r, w1_r, b1_r, w2_r, b2_r, w3_r, b3_r, e_r, o_r):
        v = jax.nn.relu(_mmb(h_r[...], w1_r[...]) + b1_r[0:1, :])
        v = jax.nn.relu(_mmb(v, w2_r[...]) + b2_r[0:1, :])
        v = (_mmb(v, w3_r[...]) + b3_r[0:1, :]) + e_r[0:1, :]
        o_r[...] = v

    return pl.pallas_call(
        body,
        in_specs=[pl.BlockSpec(a.shape, lambda: (0,) * a.ndim)
                  for a in (h, fc1, b1, fc2, b2, fc3, b3, eyef)],
        out_specs=pl.BlockSpec((8, 144), lambda: (0, 0)),
        out_shape=jax.ShapeDtypeStruct((8, 144), jnp.float32),
    )(h, fc1, b1, fc2, b2, fc3, b3, eyef)


def _apply_trans(xt, trans, p=1000):
    n, c = xt.shape

    def body(x_r, t_r, o_r):
        o_r[...] = _mmb(x_r[...], t_r[...])

    return pl.pallas_call(
        body,
        grid=(n // p,),
        in_specs=[pl.BlockSpec((p, c), lambda i: (i, 0)),
                  pl.BlockSpec(trans.shape, lambda i: (0, 0))],
        out_specs=pl.BlockSpec((p, c), lambda i: (i, 0)),
        out_shape=jax.ShapeDtypeStruct((n, c), jnp.float32),
    )(xt, trans)


def _stn_point_major(xt, p, pre):
    w1 = jnp.pad(p[pre + '_c1'].T, ((0, 4), (0, 0)))
    y1, s, q = _mm_stats([xt], [w1], bias=_brow(p[pre + '_c1b']))
    y2, s2, q2 = _mm_stats([y1], [p[pre + '_c2'].T], nrms=[_mu_inv(s, q, 8000.0)],
                           act='relu', bias=_brow(p[pre + '_c2b']))
    y3, s3, q3 = _mm_stats([y2], [p[pre + '_c3'].T], nrms=[_mu_inv(s2, q2, 8000.0)],
                           act='relu', bias=_brow(p[pre + '_c3b']))
    h = _colmax(y3, *_mu_inv(s3, q3, 8000.0))
    eyef = _brow(jnp.eye(12, dtype=jnp.float32).reshape(144))
    t = _stn_fc(h, p[pre + '_fc1'].T, _brow(p[pre + '_fc1b']),
                p[pre + '_fc2'].T, _brow(p[pre + '_fc2b']),
                p[pre + '_fc3'].T, _brow(p[pre + '_fc3b']), eyef)
    trans = jnp.pad(t[0].reshape(12, 12), ((0, 4), (0, 4)))
    return _apply_trans(xt, trans)


def _brow(v):
    return jnp.broadcast_to(v[None, :], (8, v.shape[0]))


def _head_mid(y4c, y4n, mu_c, inv_c, mu_n, inv_n, fa_l, fa_r, p=1000):
    n = y4c.shape[0]
    f32 = jnp.float32

    def body(yc_r, yn_r, mc_r, ic_r, mn_r, in_r, fl_r, fr_r,
             hc_r, hn_r, yfa_r, s_r, q_r):
        ca = jax.nn.leaky_relu((yc_r[...] - mc_r[0:1, :]) * ic_r[0:1, :], 0.2)
        na = jax.nn.leaky_relu((yn_r[...] - mn_r[0:1, :]) * in_r[0:1, :], 0.2)
        avg_c = jnp.sum(ca, axis=1, keepdims=True) / 512.0
        avg_n = jnp.sum(na, axis=1, keepdims=True) / 512.0
        avg = avg_c + avg_n
        hc = ca * (avg_c / avg)
        hn = na * (avg_n / avg)
        hc_r[...] = hc
        hn_r[...] = hn
        yfa = _mmb(hc, fl_r[...]) + _mmb(hn, fr_r[...])
        yfa_r[...] = yfa

        @pl.when(pl.program_id(0) == 0)
        def _():
            s_r[...] = jnp.zeros_like(s_r)
            q_r[...] = jnp.zeros_like(q_r)

        s_r[0:1, :] = s_r[0:1, :] + jnp.sum(yfa, axis=0, keepdims=True)
        q_r[0:1, :] = q_r[0:1, :] + jnp.sum(yfa * yfa, axis=0, keepdims=True)

    return pl.pallas_call(
        body,
        grid=(n // p,),
        in_specs=[pl.BlockSpec((p, 256), lambda i: (i, 0)),
                  pl.BlockSpec((p, 256), lambda i: (i, 0))] +
                 [pl.BlockSpec((8, 256), lambda i: (0, 0))] * 4 +
                 [pl.BlockSpec((256, 512), lambda i: (0, 0))] * 2,
        out_specs=[pl.BlockSpec((p, 256), lambda i: (i, 0)),
                   pl.BlockSpec((p, 256), lambda i: (i, 0)),
                   pl.BlockSpec((p, 512), lambda i: (i, 0)),
                   pl.BlockSpec((8, 512), lambda i: (0, 0)),
                   pl.BlockSpec((8, 512), lambda i: (0, 0))],
        out_shape=[jax.ShapeDtypeStruct((n, 256), f32),
                   jax.ShapeDtypeStruct((n, 256), f32),
                   jax.ShapeDtypeStruct((n, 512), f32),
                   jax.ShapeDtypeStruct((8, 512), f32),
                   jax.ShapeDtypeStruct((8, 512), f32)],
    )(y4c, y4n, mu_c, inv_c, mu_n, inv_n, fa_l, fa_r)


def _head_fa(hc, hn, yfa, mu, inv, p1l, p1r, p=1000):
    n = hc.shape[0]
    f32 = jnp.float32

    def body(hc_r, hn_r, yfa_r, m_r, i_r, wl_r, wr_r, y_r, s_r, q_r):
        w = jax.nn.leaky_relu((yfa_r[...] - m_r[0:1, :]) * i_r[0:1, :], 0.2)
        h2c = w[:, 0:256] * hc_r[...]
        h2n = w[:, 256:512] * hn_r[...]
        y = _mmb(h2c, wl_r[...]) + _mmb(h2n, wr_r[...])
        y_r[...] = y

        @pl.when(pl.program_id(0) == 0)
        def _():
            s_r[...] = jnp.zeros_like(s_r)
            q_r[...] = jnp.zeros_like(q_r)

        s_r[0:1, :] = s_r[0:1, :] + jnp.sum(y, axis=0, keepdims=True)
        q_r[0:1, :] = q_r[0:1, :] + jnp.sum(y * y, axis=0, keepdims=True)

    return pl.pallas_call(
        body,
        grid=(n // p,),
        in_specs=[pl.BlockSpec((p, 256), lambda i: (i, 0)),
                  pl.BlockSpec((p, 256), lambda i: (i, 0)),
                  pl.BlockSpec((p, 512), lambda i: (i, 0)),
                  pl.BlockSpec((8, 512), lambda i: (0, 0)),
                  pl.BlockSpec((8, 512), lambda i: (0, 0)),
                  pl.BlockSpec((256, 512), lambda i: (0, 0)),
                  pl.BlockSpec((256, 512), lambda i: (0, 0))],
        out_specs=[pl.BlockSpec((p, 512), lambda i: (i, 0)),
                   pl.BlockSpec((8, 512), lambda i: (0, 0)),
                   pl.BlockSpec((8, 512), lambda i: (0, 0))],
        out_shape=[jax.ShapeDtypeStruct((n, 512), f32),
                   jax.ShapeDtypeStruct((8, 512), f32),
                   jax.ShapeDtypeStruct((8, 512), f32)],
    )(hc, hn, yfa, mu, inv, p1l, p1r)


def _head_final(y7, mu, inv, w4p, p=1000):
    n = y7.shape[0]

    def body(y_r, m_r, i_r, w_r, o_r):
        a = jax.nn.leaky_relu((y_r[...] - m_r[0:1, :]) * i_r[0:1, :], 0.2)
        yv = _mmb(a, w_r[...])
        s0 = yv[:, 0:1]
        s1 = yv[:, 1:2]
        mx = jnp.maximum(s0, s1)
        e0 = s0 - mx
        e1 = s1 - mx
        lse = jnp.log(jnp.exp(e0) + jnp.exp(e1))
        o_r[...] = jnp.concatenate([e0 - lse, e1 - lse], axis=1)

    return pl.pallas_call(
        body,
        grid=(n // p,),
        in_specs=[pl.BlockSpec((p, 128), lambda i: (i, 0)),
                  pl.BlockSpec((8, 128), lambda i: (0, 0)),
                  pl.BlockSpec((8, 128), lambda i: (0, 0)),
                  pl.BlockSpec((128, 128), lambda i: (0, 0))],
        out_specs=pl.BlockSpec((p, 2), lambda i: (i, 0)),
        out_shape=jax.ShapeDtypeStruct((n, 2), jnp.float32),
    )(y7, mu, inv, w4p)


def _pad_lanes(a, width):
    return jnp.pad(a, ((0, 0), (0, width - a.shape[1])))


def _edge_layer(ct, xr, tn, wconv_c, wconv_n, watt, c, cpad):
    idx = _knn_idx(ct)
    idx_t = idx.T.reshape(N_PTS * K_NN)
    gc, gx, gn = _sc_gather3(ct, xr, tn, idx_t)
    k3 = (K_NN, N_PTS, cpad)
    gc, gx, gn = gc.reshape(k3), gx.reshape(k3), gn.reshape(k3)

    def split(w):
        wl = jnp.pad(w[:, :c].T, ((0, cpad - c), (0, 0)))
        wr = jnp.pad(w[:, c:].T, ((0, cpad - c), (0, 0)))
        return wl, wr
    w1nb, w1cen = split(wconv_c)
    w2nb, w2cen = split(wconv_n)
    w3l, w3r = split(watt)

    y1, y2, y3, s1, q1, s2, q2, s3, q3 = _edge_conv_stats(
        gc, gx, gn, ct, xr, tn, w1nb, w1cen, w2nb, w2cen, w3l, w3r)
    mu1, inv1 = _mu_inv(s1, q1)
    mu2, inv2 = _mu_inv(s2, q2)
    mu3, inv3 = _mu_inv(s3, q3)
    return _edge_attn(y1, y2, y3, mu1, inv1, mu2, inv2, mu3, inv3)


def kernel(x, att1_c, att2_c, att3_c, conv1_c, conv1_n, conv2_c, conv2_n,
           conv3_c, conv3_n, conv4_c, conv4_n, fa, pred1, pred2, pred3, pred4,
           stnc_c1, stnc_c1b, stnc_c2, stnc_c2b, stnc_c3, stnc_c3b,
           stnc_fc1, stnc_fc1b, stnc_fc2, stnc_fc2b, stnc_fc3, stnc_fc3b,
           stnn_c1, stnn_c1b, stnn_c2, stnn_c2b, stnn_c3, stnn_c3b,
           stnn_fc1, stnn_fc1b, stnn_fc2, stnn_fc2b, stnn_fc3, stnn_fc3b):
    p = dict(
        stnc_c1=stnc_c1, stnc_c1b=stnc_c1b, stnc_c2=stnc_c2, stnc_c2b=stnc_c2b,
        stnc_c3=stnc_c3, stnc_c3b=stnc_c3b, stnc_fc1=stnc_fc1,
        stnc_fc1b=stnc_fc1b, stnc_fc2=stnc_fc2, stnc_fc2b=stnc_fc2b,
        stnc_fc3=stnc_fc3, stnc_fc3b=stnc_fc3b,
        stnn_c1=stnn_c1, stnn_c1b=stnn_c1b, stnn_c2=stnn_c2, stnn_c2b=stnn_c2b,
        stnn_c3=stnn_c3, stnn_c3b=stnn_c3b, stnn_fc1=stnn_fc1,
        stnn_fc1b=stnn_fc1b, stnn_fc2=stnn_fc2, stnn_fc2b=stnn_fc2b,
        stnn_fc3=stnn_fc3, stnn_fc3b=stnn_fc3b,
    )
    B, _, N = x.shape
    coor = x[:, :12, :]
    nor = x[:, 12:, :]
    trans_c = _stnkd(coor, p, 'stnc', 12)
    coor = jnp.einsum('bcn,bcd->bdn', coor, trans_c)
    trans_n = _stnkd(nor, p, 'stnn', 12)
    nor = jnp.einsum('bcn,bcd->bdn', nor, trans_n)
    ct1 = _pad_lanes(coor[0].T, 16)
    tn1 = _pad_lanes(nor[0].T, 16)
    xr1 = _pad_lanes(coor.reshape(N, 12), 16)
    co1, no1 = _edge_layer(ct1, xr1, tn1, conv1_c, conv1_n, att1_c, 12, 16)

    xr2 = co1.T.reshape(N, 32)
    co2, no2 = _edge_layer(co1, xr2, no1, conv2_c, conv2_n, att2_c, 32, 32)

    xr3 = co2.T.reshape(N, 64)
    co3, no3 = _edge_layer(co2, xr3, no2, conv3_c, conv3_n, att3_c, 64, 64)

    y4c, s4c, q4c = _mm_stats(
        [co1, co2, co3],
        [conv4_c[:, :32].T, conv4_c[:, 32:96].T, conv4_c[:, 96:].T])
    y4n, s4n, q4n = _mm_stats(
        [no1, no2, no3],
        [conv4_n[:, :32].T, conv4_n[:, 32:96].T, conv4_n[:, 96:].T])
    mu_c, inv_c = _mu_inv(s4c, q4c, 8000.0)
    mu_n, inv_n = _mu_inv(s4n, q4n, 8000.0)
    hc, hn, yfa, sfa, qfa = _head_mid(y4c, y4n, mu_c, inv_c, mu_n, inv_n,
                                      fa[:, :256].T, fa[:, 256:].T)
    mu_f, inv_f = _mu_inv(sfa, qfa, 8000.0)
    y5, s5, q5 = _head_fa(hc, hn, yfa, mu_f, inv_f,
                          pred1[:, :256].T, pred1[:, 256:].T)
    y6, s6, q6 = _mm_stats([y5], [pred2.T], nrms=[_mu_inv(s5, q5, 8000.0)],
                           act='lrelu')
    y7, s7, q7 = _mm_stats([y6], [pred3.T], nrms=[_mu_inv(s6, q6, 8000.0)],
                           act='lrelu')
    w4p = jnp.pad(pred4.T, ((0, 0), (0, 126)))
    out = _head_final(y7, *_mu_inv(s7, q7, 8000.0), w4p)
    return out[None]


# speedup vs baseline: 1.0763x; 1.0141x over previous
import functools

import jax
import jax.numpy as jnp
from jax import lax
from jax.experimental import pallas as pl
from jax.experimental.pallas import tpu as pltpu
from jax.experimental.pallas import tpu_sc as plsc

K_NN = 16
N_PTS = 8000
NPAD = 8192
ROWS = 256


def _knn_body(xa_ref, xbt_ref, xxr_ref, xxc_ref, out_ref):
    m = jnp.dot(xa_ref[...].astype(jnp.bfloat16),
                xbt_ref[...].astype(jnp.bfloat16),
                preferred_element_type=jnp.float32)
    inner = -2.0 * m
    s = (-xxr_ref[:, 0:1] - inner) - xxc_ref[0:1, :]
    col = lax.broadcasted_iota(jnp.int32, (ROWS, NPAD), 1)
    neg = jnp.float32(-jnp.inf)
    s = jnp.where(col < N_PTS, s, neg)
    for t in range(K_NN + 1):
        idx = jnp.argmax(s, axis=1).astype(jnp.int32)[:, None]
        out_ref[:, t : t + 1] = idx
        s = jnp.where(col == idx, neg, s)


def _knn_pallas(xa, xbt, xxr, xxc):
    return pl.pallas_call(
        _knn_body,
        grid=(NPAD // ROWS,),
        in_specs=[
            pl.BlockSpec((ROWS, 128), lambda i: (i, 0)),
            pl.BlockSpec((128, NPAD), lambda i: (0, 0)),
            pl.BlockSpec((ROWS, 8), lambda i: (i, 0)),
            pl.BlockSpec((8, NPAD), lambda i: (0, 0)),
        ],
        out_specs=pl.BlockSpec((ROWS, 32), lambda i: (i, 0)),
        out_shape=jax.ShapeDtypeStruct((NPAD, 32), jnp.int32),
    )(xa, xbt, xxr, xxc)


def _knn_idx(xt):
    n, c = xt.shape
    xx = jnp.sum(xt.T[None] * xt.T[None], axis=1)[0]
    xpad = jnp.pad(xt, ((0, NPAD - n), (0, 128 - c)))
    xxp = jnp.pad(xx, (0, NPAD - n))
    xxr = jnp.broadcast_to(xxp[:, None], (NPAD, 8))
    xxc = jnp.broadcast_to(xxp[None, :], (8, NPAD))
    out = _knn_pallas(xpad, xpad.T, xxr, xxc)
    return out[:N_PTS, 1 : K_NN + 1]


def _sc_gather3(t1, t2, t3, idx_flat):
    b = idx_flat.shape[0]
    info = plsc.get_sparse_core_info()
    nw = info.num_cores * info.num_subcores
    b_per_w = b // nw
    ch = 80
    n_iter = b_per_w // ch
    mesh = plsc.VectorSubcoreMesh(core_axis_name="c", subcore_axis_name="s")
    d1, d2, d3 = t1.shape[1], t2.shape[1], t3.shape[1]

    @functools.partial(
        pl.kernel,
        mesh=mesh,
        compiler_params=pltpu.CompilerParams(use_tc_tiling_on_sc=False),
        out_type=(
            jax.ShapeDtypeStruct((b, d1), jnp.float32),
            jax.ShapeDtypeStruct((b, d2), jnp.float32),
            jax.ShapeDtypeStruct((b, d3), jnp.float32),
        ),
        scratch_types=[
            pltpu.VMEM((ch,), jnp.int32),
            pltpu.VMEM((ch, d1), jnp.float32),
            pltpu.VMEM((ch, d2), jnp.float32),
            pltpu.VMEM((ch, d3), jnp.float32),
            pltpu.SemaphoreType.DMA,
        ],
    )
    def gat(t1_hbm, t2_hbm, t3_hbm, idx_hbm, o1_hbm, o2_hbm, o3_hbm,
            idx_v, r1_v, r2_v, r3_v, sem):
        wid = lax.axis_index("s") * info.num_cores + lax.axis_index("c")
        base = wid * b_per_w

        def body(j, carry):
            off = base + j * ch
            pltpu.sync_copy(idx_hbm.at[pl.ds(off, ch)], idx_v)
            c1 = pltpu.async_copy(t1_hbm.at[idx_v], r1_v, sem)
            c2 = pltpu.async_copy(t2_hbm.at[idx_v], r2_v, sem)
            c3 = pltpu.async_copy(t3_hbm.at[idx_v], r3_v, sem)
            c1.wait()
            c2.wait()
            c3.wait()
            pltpu.sync_copy(r1_v, o1_hbm.at[pl.ds(off, ch)])
            pltpu.sync_copy(r2_v, o2_hbm.at[pl.ds(off, ch)])
            pltpu.sync_copy(r3_v, o3_hbm.at[pl.ds(off, ch)])
            return carry

        lax.fori_loop(0, n_iter, body, 0)

    return gat(t1, t2, t3, idx_flat)


def _lrelu(x):
    return jax.nn.leaky_relu(x, 0.2)


def _bn(x):
    axes = (0,) + tuple(range(2, x.ndim))
    m = jnp.mean(x, axis=axes, keepdims=True)
    v = jnp.var(x, axis=axes, keepdims=True)
    return (x - m) * lax.rsqrt(v + 1e-5)


def _conv1d(x, w, bias=None):
    y = jnp.einsum('oc,bcn->bon', w, x)
    if bias is not None:
        y = y + bias[None, :, None]
    return y


def _stnkd(x, p, pre, k):
    h = jax.nn.relu(_bn(_conv1d(x, p[pre + '_c1'], p[pre + '_c1b'])))
    h = jax.nn.relu(_bn(_conv1d(h, p[pre + '_c2'], p[pre + '_c2b'])))
    h = jax.nn.relu(_bn(_conv1d(h, p[pre + '_c3'], p[pre + '_c3b'])))
    h = jnp.max(h, axis=2)
    h = jax.nn.relu(h @ p[pre + '_fc1'].T + p[pre + '_fc1b'])
    h = jax.nn.relu(h @ p[pre + '_fc2'].T + p[pre + '_fc2b'])
    h = h @ p[pre + '_fc3'].T + p[pre + '_fc3b']
    h = h + jnp.eye(k, dtype=h.dtype).reshape(1, k * k)
    return h.reshape(-1, k, k)


def _mmb(x, w):
    return jnp.dot(x.astype(jnp.bfloat16), w.astype(jnp.bfloat16),
                   preferred_element_type=jnp.float32)


def _edge_conv_stats(gc, gx, gn, tc, tx, tn, w1nb, w1cen, w2nb, w2cen, w3l, w3r):
    k = K_NN
    n = N_PTS
    p = 2000
    cc = gc.shape[2]
    cn = gn.shape[2]
    co1, co2, co3 = w1nb.shape[1], w2nb.shape[1], w3l.shape[1]

    def body(gc_r, gx_r, gn_r, tc_r, tx_r, tn_r, w1a_r, w1b_r, w2a_r, w2b_r,
             w3a_r, w3b_r, y1_r, y2_r, y3_r, s1_r, q1_r, s2_r, q2_r, s3_r, q3_r):
        gcb, gxb, gnb = gc_r[0], gx_r[0], gn_r[0]
        y1 = _mmb(gcb, w1a_r[...]) + _mmb(tc_r[...], w1b_r[...])
        y2 = _mmb(gnb, w2a_r[...]) + _mmb(tn_r[...], w2b_r[...])
        delta = tx_r[...] - gxb
        y3 = _mmb(delta, w3a_r[...]) + _mmb(gxb, w3b_r[...])
        y1_r[0] = y1
        y2_r[0] = y2
        y3_r[0] = y3
        first = (pl.program_id(0) == 0) & (pl.program_id(1) == 0)

        @pl.when(first)
        def _():
            s1_r[...] = jnp.zeros_like(s1_r)
            q1_r[...] = jnp.zeros_like(q1_r)
            s2_r[...] = jnp.zeros_like(s2_r)
            q2_r[...] = jnp.zeros_like(q2_r)
            s3_r[...] = jnp.zeros_like(s3_r)
            q3_r[...] = jnp.zeros_like(q3_r)

        for y, s_r, q_r in ((y1, s1_r, q1_r), (y2, s2_r, q2_r), (y3, s3_r, q3_r)):
            s_r[0:1, :] = s_r[0:1, :] + jnp.sum(y, axis=0, keepdims=True)
            q_r[0:1, :] = q_r[0:1, :] + jnp.sum(y * y, axis=0, keepdims=True)

    f32 = jnp.float32
    return pl.pallas_call(
        body,
        grid=(k, n // p),
        in_specs=[
            pl.BlockSpec((1, p, cc), lambda a, i: (a, i, 0)),
            pl.BlockSpec((1, p, cc), lambda a, i: (a, i, 0)),
            pl.BlockSpec((1, p, cn), lambda a, i: (a, i, 0)),
            pl.BlockSpec((p, cc), lambda a, i: (i, 0)),
            pl.BlockSpec((p, cc), lambda a, i: (i, 0)),
            pl.BlockSpec((p, cn), lambda a, i: (i, 0)),
            pl.BlockSpec((cc, co1), lambda a, i: (0, 0)),
            pl.BlockSpec((cc, co1), lambda a, i: (0, 0)),
            pl.BlockSpec((cn, co2), lambda a, i: (0, 0)),
            pl.BlockSpec((cn, co2), lambda a, i: (0, 0)),
            pl.BlockSpec((cc, co3), lambda a, i: (0, 0)),
            pl.BlockSpec((cc, co3), lambda a, i: (0, 0)),
        ],
        out_specs=[
            pl.BlockSpec((1, p, co1), lambda a, i: (a, i, 0)),
            pl.BlockSpec((1, p, co2), lambda a, i: (a, i, 0)),
            pl.BlockSpec((1, p, co3), lambda a, i: (a, i, 0)),
            pl.BlockSpec((8, co1), lambda a, i: (0, 0)),
            pl.BlockSpec((8, co1), lambda a, i: (0, 0)),
            pl.BlockSpec((8, co2), lambda a, i: (0, 0)),
            pl.BlockSpec((8, co2), lambda a, i: (0, 0)),
            pl.BlockSpec((8, co3), lambda a, i: (0, 0)),
            pl.BlockSpec((8, co3), lambda a, i: (0, 0)),
        ],
        out_shape=[
            jax.ShapeDtypeStruct((k, n, co1), f32),
            jax.ShapeDtypeStruct((k, n, co2), f32),
            jax.ShapeDtypeStruct((k, n, co3), f32),
            jax.ShapeDtypeStruct((8, co1), f32),
            jax.ShapeDtypeStruct((8, co1), f32),
            jax.ShapeDtypeStruct((8, co2), f32),
            jax.ShapeDtypeStruct((8, co2), f32),
            jax.ShapeDtypeStruct((8, co3), f32),
            jax.ShapeDtypeStruct((8, co3), f32),
        ],
    )(gc, gx, gn, tc, tx, tn, w1nb, w1cen, w2nb, w2cen, w3l, w3r)


def _edge_attn(y1, y2, y3, mu1, inv1, mu2, inv2, mu3, inv3):
    k = K_NN
    n = N_PTS
    p = 400
    co1 = y1.shape[2]
    co2 = y2.shape[2]

    def body(y1_r, y2_r, y3_r, m1_r, i1_r, m2_r, i2_r, m3_r, i3_r,
             co_r, no_r, wb, fb):
        m1, i1 = m1_r[0:1, :], i1_r[0:1, :]
        m2, i2 = m2_r[0:1, :], i2_r[0:1, :]
        m3, i3 = m3_r[0:1, :], i3_r[0:1, :]
        emax = None
        nmax = None
        for a in range(K_NN):
            e = jax.nn.leaky_relu((y3_r[a] - m3) * i3, 0.2)
            wb[a] = e
            emax = e if a == 0 else jnp.maximum(emax, e)
            fb[a] = jax.nn.leaky_relu((y1_r[a] - m1) * i1, 0.2)
            nk = jax.nn.leaky_relu((y2_r[a] - m2) * i2, 0.2)
            nmax = nk if a == 0 else jnp.maximum(nmax, nk)
        den = jnp.zeros_like(emax)
        for a in range(K_NN):
            wv = jnp.exp(wb[a] - emax)
            wb[a] = wv
            den = den + wv
        acc = jnp.zeros_like(emax)
        for a in range(K_NN):
            acc = acc + (wb[a] / den) * fb[a]
        co_r[...] = acc
        no_r[...] = nmax

    f32 = jnp.float32
    return pl.pallas_call(
        body,
        grid=(n // p,),
        in_specs=[
            pl.BlockSpec((k, p, co1), lambda i: (0, i, 0)),
            pl.BlockSpec((k, p, co2), lambda i: (0, i, 0)),
            pl.BlockSpec((k, p, co1), lambda i: (0, i, 0)),
            pl.BlockSpec((8, co1), lambda i: (0, 0)),
            pl.BlockSpec((8, co1), lambda i: (0, 0)),
            pl.BlockSpec((8, co2), lambda i: (0, 0)),
            pl.BlockSpec((8, co2), lambda i: (0, 0)),
            pl.BlockSpec((8, co1), lambda i: (0, 0)),
            pl.BlockSpec((8, co1), lambda i: (0, 0)),
        ],
        out_specs=[
            pl.BlockSpec((p, co1), lambda i: (i, 0)),
            pl.BlockSpec((p, co2), lambda i: (i, 0)),
        ],
        out_shape=[
            jax.ShapeDtypeStruct((n, co1), f32),
            jax.ShapeDtypeStruct((n, co2), f32),
        ],
        scratch_shapes=[
            pltpu.VMEM((k, p, co1), f32),
            pltpu.VMEM((k, p, co1), f32),
        ],
    )(y1, y2, y3, mu1, inv1, mu2, inv2, mu3, inv3)


def _mu_inv(s, q, m=float(N_PTS * K_NN)):
    mu = s[0:1, :] / m
    var = q[0:1, :] / m - mu * mu
    inv = lax.rsqrt(var + 1e-5)
    return (jnp.broadcast_to(mu, (8,) + mu.shape[1:]),
            jnp.broadcast_to(inv, (8,) + inv.shape[1:]))


def _act(v, act):
    if act == 'relu':
        return jax.nn.relu(v)
    if act == 'lrelu':
        return jax.nn.leaky_relu(v, 0.2)
    return v


def _mm_stats(xs, ws, nrms=None, act=None, bias=None, p=1000):
    n = xs[0].shape[0]
    co = ws[0].shape[1]
    nx = len(xs)
    f32 = jnp.float32

    def body(*refs):
        x_rs = refs[:nx]
        pos = nx
        n_rs = None
        if nrms is not None:
            n_rs = refs[pos:pos + 2 * nx]
            pos += 2 * nx
        w_rs = refs[pos:pos + nx]
        pos += nx
        b_r = refs[pos] if bias is not None else None
        pos += 1 if bias is not None else 0
        y_r, s_r, q_r = refs[pos:pos + 3]
        y = None
        for i in range(nx):
            xv = x_rs[i][...]
            if n_rs is not None:
                xv = _act((xv - n_rs[2 * i][0:1, :]) * n_rs[2 * i + 1][0:1, :], act)
            t = _mmb(xv, w_rs[i][...])
            y = t if y is None else y + t
        if b_r is not None:
            y = y + b_r[0:1, :]
        y_r[...] = y

        @pl.when(pl.program_id(0) == 0)
        def _():
            s_r[...] = jnp.zeros_like(s_r)
            q_r[...] = jnp.zeros_like(q_r)

        s_r[0:1, :] = s_r[0:1, :] + jnp.sum(y, axis=0, keepdims=True)
        q_r[0:1, :] = q_r[0:1, :] + jnp.sum(y * y, axis=0, keepdims=True)

    in_specs = [pl.BlockSpec((p, x.shape[1]), lambda i: (i, 0)) for x in xs]
    args = list(xs)
    if nrms is not None:
        for mu, inv in nrms:
            args += [mu, inv]
            in_specs += [pl.BlockSpec(mu.shape, lambda i: (0, 0)),
                         pl.BlockSpec(inv.shape, lambda i: (0, 0))]
    args += list(ws)
    in_specs += [pl.BlockSpec(w.shape, lambda i: (0, 0)) for w in ws]
    if bias is not None:
        args.append(bias)
        in_specs.append(pl.BlockSpec(bias.shape, lambda i: (0, 0)))
    return pl.pallas_call(
        body,
        grid=(n // p,),
        in_specs=in_specs,
        out_specs=[pl.BlockSpec((p, co), lambda i: (i, 0)),
                   pl.BlockSpec((8, co), lambda i: (0, 0)),
                   pl.BlockSpec((8, co), lambda i: (0, 0))],
        out_shape=[jax.ShapeDtypeStruct((n, co), f32),
                   jax.ShapeDtypeStruct((8, co), f32),
                   jax.ShapeDtypeStruct((8, co), f32)],
    )(*args)


def _colmax(x, mu, inv, p=1000):
    n, c = x.shape

    def body(x_r, mu_r, inv_r, o_r):
        v = jax.nn.relu((x_r[...] - mu_r[0:1, :]) * inv_r[0:1, :])
        m = jnp.max(v, axis=0, keepdims=True)

        @pl.when(pl.program_id(0) == 0)
        def _():
            o_r[...] = jnp.full_like(o_r, -jnp.inf)

        o_r[...] = jnp.maximum(o_r[...], m)

    return pl.pallas_call(
        body,
        grid=(n // p,),
        in_specs=[pl.BlockSpec((p, c), lambda i: (i, 0)),
                  pl.BlockSpec((8, c), lambda i: (0, 0)),
                  pl.BlockSpec((8, c), lambda i: (0, 0))],
        out_specs=pl.BlockSpec((8, c), lambda i: (0, 0)),
        out_shape=jax.ShapeDtypeStruct((8, c), jnp.float32),
    )(x, mu, inv)


def _stn_fc(h, fc1, b1, fc2, b2, fc3, b3, eyef):
    def body(h_r, w1_r, b1_r, w2_r, b2_r, w3_r, b3_r, e_r, o_r):
        v = jax.nn.relu(_mmb(h_r[...], w1_r[...]) + b1_r[0:1, :])
        v = jax.nn.relu(_mmb(v, w2_r[...]) + b2_r[0:1, :])
        v = (_mmb(v, w3_r[...]) + b3_r[0:1, :]) + e_r[0:1, :]
        o_r[...] = v

    return pl.pallas_call(
        body,
        in_specs=[pl.BlockSpec(a.shape, lambda: (0,) * a.ndim)
                  for a in (h, fc1, b1, fc2, b2, fc3, b3, eyef)],
        out_specs=pl.BlockSpec((8, 144), lambda: (0, 0)),
        out_shape=jax.ShapeDtypeStruct((8, 144), jnp.float32),
    )(h, fc1, b1, fc2, b2, fc3, b3, eyef)


def _apply_trans(xt, trans, p=1000):
    n, c = xt.shape

    def body(x_r, t_r, o_r):
        o_r[...] = _mmb(x_r[...], t_r[...])

    return pl.pallas_call(
        body,
        grid=(n // p,),
        in_specs=[pl.BlockSpec((p, c), lambda i: (i, 0)),
                  pl.BlockSpec(trans.shape, lambda i: (0, 0))],
        out_specs=pl.BlockSpec((p, c), lambda i: (i, 0)),
        out_shape=jax.ShapeDtypeStruct((n, c), jnp.float32),
    )(xt, trans)


def _stn_point_major(xt, p, pre):
    w1 = jnp.pad(p[pre + '_c1'].T, ((0, 4), (0, 0)))
    y1, s, q = _mm_stats([xt], [w1], bias=_brow(p[pre + '_c1b']))
    y2, s2, q2 = _mm_stats([y1], [p[pre + '_c2'].T], nrms=[_mu_inv(s, q, 8000.0)],
                           act='relu', bias=_brow(p[pre + '_c2b']))
    y3, s3, q3 = _mm_stats([y2], [p[pre + '_c3'].T], nrms=[_mu_inv(s2, q2, 8000.0)],
                           act='relu', bias=_brow(p[pre + '_c3b']))
    h = _colmax(y3, *_mu_inv(s3, q3, 8000.0))
    eyef = _brow(jnp.eye(12, dtype=jnp.float32).reshape(144))
    t = _stn_fc(h, p[pre + '_fc1'].T, _brow(p[pre + '_fc1b']),
                p[pre + '_fc2'].T, _brow(p[pre + '_fc2b']),
                p[pre + '_fc3'].T, _brow(p[pre + '_fc3b']), eyef)
    trans = jnp.pad(t[0].reshape(12, 12), ((0, 4), (0, 4)))
    return _apply_trans(xt, trans)


def _brow(v):
    return jnp.broadcast_to(v[None, :], (8, v.shape[0]))


def _head_mid(y4c, y4n, mu_c, inv_c, mu_n, inv_n, fa_l, fa_r, p=1000):
    n = y4c.shape[0]
    f32 = jnp.float32

    def body(yc_r, yn_r, mc_r, ic_r, mn_r, in_r, fl_r, fr_r,
             hc_r, hn_r, yfa_r, s_r, q_r):
        ca = jax.nn.leaky_relu((yc_r[...] - mc_r[0:1, :]) * ic_r[0:1, :], 0.2)
        na = jax.nn.leaky_relu((yn_r[...] - mn_r[0:1, :]) * in_r[0:1, :], 0.2)
        avg_c = jnp.sum(ca, axis=1, keepdims=True) / 512.0
        avg_n = jnp.sum(na, axis=1, keepdims=True) / 512.0
        avg = avg_c + avg_n
        hc = ca * (avg_c / avg)
        hn = na * (avg_n / avg)
        hc_r[...] = hc
        hn_r[...] = hn
        yfa = _mmb(hc, fl_r[...]) + _mmb(hn, fr_r[...])
        yfa_r[...] = yfa

        @pl.when(pl.program_id(0) == 0)
        def _():
            s_r[...] = jnp.zeros_like(s_r)
            q_r[...] = jnp.zeros_like(q_r)

        s_r[0:1, :] = s_r[0:1, :] + jnp.sum(yfa, axis=0, keepdims=True)
        q_r[0:1, :] = q_r[0:1, :] + jnp.sum(yfa * yfa, axis=0, keepdims=True)

    return pl.pallas_call(
        body,
        grid=(n // p,),
        in_specs=[pl.BlockSpec((p, 256), lambda i: (i, 0)),
                  pl.BlockSpec((p, 256), lambda i: (i, 0))] +
                 [pl.BlockSpec((8, 256), lambda i: (0, 0))] * 4 +
                 [pl.BlockSpec((256, 512), lambda i: (0, 0))] * 2,
        out_specs=[pl.BlockSpec((p, 256), lambda i: (i, 0)),
                   pl.BlockSpec((p, 256), lambda i: (i, 0)),
                   pl.BlockSpec((p, 512), lambda i: (i, 0)),
                   pl.BlockSpec((8, 512), lambda i: (0, 0)),
                   pl.BlockSpec((8, 512), lambda i: (0, 0))],
        out_shape=[jax.ShapeDtypeStruct((n, 256), f32),
                   jax.ShapeDtypeStruct((n, 256), f32),
                   jax.ShapeDtypeStruct((n, 512), f32),
                   jax.ShapeDtypeStruct((8, 512), f32),
                   jax.ShapeDtypeStruct((8, 512), f32)],
    )(y4c, y4n, mu_c, inv_c, mu_n, inv_n, fa_l, fa_r)


def _head_fa(hc, hn, yfa, mu, inv, p1l, p1r, p=1000):
    n = hc.shape[0]
    f32 = jnp.float32

    def body(hc_r, hn_r, yfa_r, m_r, i_r, wl_r, wr_r, y_r, s_r, q_r):
        w = jax.nn.leaky_relu((yfa_r[...] - m_r[0:1, :]) * i_r[0:1, :], 0.2)
        h2c = w[:, 0:256] * hc_r[...]
        h2n = w[:, 256:512] * hn_r[...]
        y = _mmb(h2c, wl_r[...]) + _mmb(h2n, wr_r[...])
        y_r[...] = y

        @pl.when(pl.program_id(0) == 0)
        def _():
            s_r[...] = jnp.zeros_like(s_r)
            q_r[...] = jnp.zeros_like(q_r)

        s_r[0:1, :] = s_r[0:1, :] + jnp.sum(y, axis=0, keepdims=True)
        q_r[0:1, :] = q_r[0:1, :] + jnp.sum(y * y, axis=0, keepdims=True)

    return pl.pallas_call(
        body,
        grid=(n // p,),
        in_specs=[pl.BlockSpec((p, 256), lambda i: (i, 0)),
                  pl.BlockSpec((p, 256), lambda i: (i, 0)),
                  pl.BlockSpec((p, 512), lambda i: (i, 0)),
                  pl.BlockSpec((8, 512), lambda i: (0, 0)),
                  pl.BlockSpec((8, 512), lambda i: (0, 0)),
                  pl.BlockSpec((256, 512), lambda i: (0, 0)),
                  pl.BlockSpec((256, 512), lambda i: (0, 0))],
        out_specs=[pl.BlockSpec((p, 512), lambda i: (i, 0)),
                   pl.BlockSpec((8, 512), lambda i: (0, 0)),
                   pl.BlockSpec((8, 512), lambda i: (0, 0))],
        out_shape=[jax.ShapeDtypeStruct((n, 512), f32),
                   jax.ShapeDtypeStruct((8, 512), f32),
                   jax.ShapeDtypeStruct((8, 512), f32)],
    )(hc, hn, yfa, mu, inv, p1l, p1r)


def _head_final(y7, mu, inv, w4p, p=1000):
    n = y7.shape[0]

    def body(y_r, m_r, i_r, w_r, o_r):
        a = jax.nn.leaky_relu((y_r[...] - m_r[0:1, :]) * i_r[0:1, :], 0.2)
        yv = _mmb(a, w_r[...])
        s0 = yv[:, 0:1]
        s1 = yv[:, 1:2]
        mx = jnp.maximum(s0, s1)
        e0 = s0 - mx
        e1 = s1 - mx
        lse = jnp.log(jnp.exp(e0) + jnp.exp(e1))
        o_r[...] = jnp.concatenate([e0 - lse, e1 - lse], axis=1)

    return pl.pallas_call(
        body,
        grid=(n // p,),
        in_specs=[pl.BlockSpec((p, 128), lambda i: (i, 0)),
                  pl.BlockSpec((8, 128), lambda i: (0, 0)),
                  pl.BlockSpec((8, 128), lambda i: (0, 0)),
                  pl.BlockSpec((128, 128), lambda i: (0, 0))],
        out_specs=pl.BlockSpec((p, 2), lambda i: (i, 0)),
        out_shape=jax.ShapeDtypeStruct((n, 2), jnp.float32),
    )(y7, mu, inv, w4p)


def _pad_lanes(a, width):
    return jnp.pad(a, ((0, 0), (0, width - a.shape[1])))


def _edge_layer(ct, xr, tn, wconv_c, wconv_n, watt, c, cpad):
    idx = _knn_idx(ct)
    idx_t = idx.T.reshape(N_PTS * K_NN)
    gc, gx, gn = _sc_gather3(ct, xr, tn, idx_t)
    k3 = (K_NN, N_PTS, cpad)
    gc, gx, gn = gc.reshape(k3), gx.reshape(k3), gn.reshape(k3)

    def split(w):
        wl = jnp.pad(w[:, :c].T, ((0, cpad - c), (0, 0)))
        wr = jnp.pad(w[:, c:].T, ((0, cpad - c), (0, 0)))
        return wl, wr
    w1nb, w1cen = split(wconv_c)
    w2nb, w2cen = split(wconv_n)
    w3l, w3r = split(watt)

    y1, y2, y3, s1, q1, s2, q2, s3, q3 = _edge_conv_stats(
        gc, gx, gn, ct, xr, tn, w1nb, w1cen, w2nb, w2cen, w3l, w3r)
    mu1, inv1 = _mu_inv(s1, q1)
    mu2, inv2 = _mu_inv(s2, q2)
    mu3, inv3 = _mu_inv(s3, q3)
    return _edge_attn(y1, y2, y3, mu1, inv1, mu2, inv2, mu3, inv3)


def kernel(x, att1_c, att2_c, att3_c, conv1_c, conv1_n, conv2_c, conv2_n,
           conv3_c, conv3_n, conv4_c, conv4_n, fa, pred1, pred2, pred3, pred4,
           stnc_c1, stnc_c1b, stnc_c2, stnc_c2b, stnc_c3, stnc_c3b,
           stnc_fc1, stnc_fc1b, stnc_fc2, stnc_fc2b, stnc_fc3, stnc_fc3b,
           stnn_c1, stnn_c1b, stnn_c2, stnn_c2b, stnn_c3, stnn_c3b,
           stnn_fc1, stnn_fc1b, stnn_fc2, stnn_fc2b, stnn_fc3, stnn_fc3b):
    p = dict(
        stnc_c1=stnc_c1, stnc_c1b=stnc_c1b, stnc_c2=stnc_c2, stnc_c2b=stnc_c2b,
        stnc_c3=stnc_c3, stnc_c3b=stnc_c3b, stnc_fc1=stnc_fc1,
        stnc_fc1b=stnc_fc1b, stnc_fc2=stnc_fc2, stnc_fc2b=stnc_fc2b,
        stnc_fc3=stnc_fc3, stnc_fc3b=stnc_fc3b,
        stnn_c1=stnn_c1, stnn_c1b=stnn_c1b, stnn_c2=stnn_c2, stnn_c2b=stnn_c2b,
        stnn_c3=stnn_c3, stnn_c3b=stnn_c3b, stnn_fc1=stnn_fc1,
        stnn_fc1b=stnn_fc1b, stnn_fc2=stnn_fc2, stnn_fc2b=stnn_fc2b,
        stnn_fc3=stnn_fc3, stnn_fc3b=stnn_fc3b,
    )
    B, _, N = x.shape
    coor = x[:, :12, :]
    nor = x[:, 12:, :]
    trans_c = _stnkd(coor, p, 'stnc', 12)
    coor = jnp.einsum('bcn,bcd->bdn', coor, trans_c)
    trans_n = _stnkd(nor, p, 'stnn', 12)
    nor = jnp.einsum('bcn,bcd->bdn', nor, trans_n)
    ct1 = _pad_lanes(coor[0].T, 16)
    tn1 = _pad_lanes(nor[0].T, 16)
    xr1 = _pad_lanes(coor.reshape(N, 12), 16)
    co1, no1 = _edge_layer(ct1, xr1, tn1, conv1_c, conv1_n, att1_c, 12, 16)

    xr2 = co1.T.reshape(N, 32)
    co2, no2 = _edge_layer(co1, xr2, no1, conv2_c, conv2_n, att2_c, 32, 32)

    xr3 = co2.T.reshape(N, 64)
    co3, no3 = _edge_layer(co2, xr3, no2, conv3_c, conv3_n, att3_c, 64, 64)

    y4c, s4c, q4c = _mm_stats(
        [co1, co2, co3],
        [conv4_c[:, :32].T, conv4_c[:, 32:96].T, conv4_c[:, 96:].T])
    y4n, s4n, q4n = _mm_stats(
        [no1, no2, no3],
        [conv4_n[:, :32].T, conv4_n[:, 32:96].T, conv4_n[:, 96:].T])
    mu_c, inv_c = _mu_inv(s4c, q4c, 8000.0)
    mu_n, inv_n = _mu_inv(s4n, q4n, 8000.0)
    hc, hn, yfa, sfa, qfa = _head_mid(y4c, y4n, mu_c, inv_c, mu_n, inv_n,
                                      fa[:, :256].T, fa[:, 256:].T)
    mu_f, inv_f = _mu_inv(sfa, qfa, 8000.0)
    y5, s5, q5 = _head_fa(hc, hn, yfa, mu_f, inv_f,
                          pred1[:, :256].T, pred1[:, 256:].T)
    y6, s6, q6 = _mm_stats([y5], [pred2.T], nrms=[_mu_inv(s5, q5, 8000.0)],
                           act='lrelu')
    y7, s7, q7 = _mm_stats([y6], [pred3.T], nrms=[_mu_inv(s6, q6, 8000.0)],
                           act='lrelu')
    w4p = jnp.pad(pred4.T, ((0, 0), (0, 126)))
    out = _head_final(y7, *_mu_inv(s7, q7, 8000.0), w4p)
    return out[None]
